# Initial kernel scaffold; baseline (speedup 1.0000x reference)
import jax
import jax.numpy as jnp
from jax.experimental import pallas as pl


def kernel(x, edge_index, batch, W1, a1s, a1d, b1, W2, a2s, a2d, b2, W3, a3s, a3d, b3, fc1_w, fc1_b, fc2_w, fc2_b):
    raise NotImplementedError("write your pallas kernel here")


# speedup vs baseline: 1.0955x; 1.0955x over previous
import functools

import jax
import jax.numpy as jnp
from jax.experimental import pallas as pl
from jax.experimental.pallas import tpu as pltpu

N = 10000
E = 320000
G = 32
H = 8


def _dense_body(x_ref, w_ref, as_ref, ad_ref, h_ref, asrc_ref, adst_ref):
    h = jnp.dot(x_ref[...], w_ref[...], preferred_element_type=jnp.float32)
    h_ref[...] = h
    asrc_ref[...] = jnp.dot(h, as_ref[...], preferred_element_type=jnp.float32)
    adst_ref[...] = jnp.dot(h, ad_ref[...], preferred_element_type=jnp.float32)


@functools.partial(jax.jit, static_argnames=("hc",))
def _dense(x, W, As, Ad, hc):
    n = x.shape[0]
    return pl.pallas_call(
        _dense_body,
        out_shape=(
            jax.ShapeDtypeStruct((n, hc), jnp.float32),
            jax.ShapeDtypeStruct((n, H), jnp.float32),
            jax.ShapeDtypeStruct((n, H), jnp.float32),
        ),
    )(x, W, As, Ad)


def _att_mat(a):
    Hh, C = a.shape
    return (jnp.eye(Hh, dtype=a.dtype)[:, None, :] * a[:, :, None]).reshape(Hh * C, Hh)


def _gat_layer_edges(h, asrc, adst, src, dst, b):
    n = h.shape[0]
    mub = asrc.max(axis=0)[None, :] + adst
    mub = jnp.where(mub > 0, mub, 0.2 * mub)
    e = asrc[src] + adst[dst]
    e = jnp.where(e > 0, e, 0.2 * e)
    w = jnp.exp(e - mub[dst])
    hc = h.shape[1]
    C = hc // H
    hsrc = h[src].reshape(-1, H, C)
    num = jax.ops.segment_sum(w[:, :, None] * hsrc, dst, num_segments=n)
    den = jax.ops.segment_sum(w, dst, num_segments=n)
    out = num / (den[:, :, None] + 1e-16)
    return out.reshape(n, hc) + b


def kernel(x, edge_index, batch, W1, a1s, a1d, b1, W2, a2s, a2d, b2,
           W3, a3s, a3d, b3, fc1_w, fc1_b, fc2_w, fc2_b):
    loop = jnp.arange(N, dtype=edge_index.dtype)
    src = jnp.concatenate([edge_index[0], loop])
    dst = jnp.concatenate([edge_index[1], loop])

    h, asrc, adst = _dense(x, W1, _att_mat(a1s), _att_mat(a1d), 64)
    h = jax.nn.elu(_gat_layer_edges(h, asrc, adst, src, dst, b1))
    h, asrc, adst = _dense(h, W2, _att_mat(a2s), _att_mat(a2d), 128)
    h = jax.nn.elu(_gat_layer_edges(h, asrc, adst, src, dst, b2))
    h, asrc, adst = _dense(h, W3, _att_mat(a3s), _att_mat(a3d), 128)
    h = jax.nn.elu(_gat_layer_edges(h, asrc, adst, src, dst, b3))

    sums = jax.ops.segment_sum(h, batch, num_segments=G)
    cnt = jax.ops.segment_sum(jnp.ones((N,), h.dtype), batch, num_segments=G)
    g = sums / jnp.maximum(cnt, 1.0)[:, None]
    z = jax.nn.relu(g @ fc1_w + fc1_b)
    return z @ fc2_w + fc2_b


# speedup vs baseline: 66.5177x; 60.7184x over previous
import functools

import jax
import jax.numpy as jnp
from jax import lax
from jax.experimental import pallas as pl
from jax.experimental.pallas import tpu as pltpu
from jax.experimental.pallas import tpu_sc as plsc

N = 10000
E = 320000
G = 32
H = 8

NC = 2
NS = 16
K = 128
NCHUNK = 81
EPW = K * NCHUNK
EPAD = EPW * NC * NS
NR = 10016
RPT = NR // NS


def _dense_body(hc, x_ref, w_ref, as_ref, ad_ref, hext_ref, dstt_ref):
    n = x_ref.shape[0]
    h = jnp.dot(x_ref[...], w_ref[...], preferred_element_type=jnp.float32)
    asrc = jnp.dot(h, as_ref[...], preferred_element_type=jnp.float32)
    adst = jnp.dot(h, ad_ref[...], preferred_element_type=jnp.float32)
    hext_ref[...] = jnp.concatenate(
        [h, asrc, jnp.zeros((n, 8), jnp.float32)], axis=1)
    m = asrc.max(axis=0)[None, :] + adst
    mub = jnp.where(m > 0, m, 0.2 * m)
    dstt_ref[...] = jnp.concatenate([adst, mub], axis=1)


@functools.partial(jax.jit, static_argnames=("hc",))
def _dense(x, W, As, Ad, hc):
    n = x.shape[0]
    row = hc + 16
    return pl.pallas_call(
        functools.partial(_dense_body, hc),
        out_shape=(
            jax.ShapeDtypeStruct((n, row), jnp.float32),
            jax.ShapeDtypeStruct((n, 16), jnp.float32),
        ),
    )(x, W, As, Ad)


def _att_mat(a):
    Hh, C = a.shape
    return (jnp.eye(Hh, dtype=a.dtype)[:, None, :] * a[:, :, None]).reshape(Hh * C, Hh)


def _sc_edge_body(hc, hext, dstt, srcp, dstp, out, src_idx, dst_idx,
                  S, Dt, R, accum, sem1, sem2):
    row = hc + 16
    nv = hc // 16
    c = lax.axis_index("c")
    s = lax.axis_index("s")
    wid = c * NS + s

    cph = hc // H
    lanes = lax.iota(jnp.int32, 16)
    idx_hi = lanes % 8 + 8
    mask_den = jnp.where(lanes < 8, 1.0, 0.0).astype(jnp.float32)

    def _zrow(k, _):
        for j in range(row // 16):
            R[k, pl.ds(16 * j, 16)] = jnp.zeros((16,), jnp.float32)
        return 0
    lax.fori_loop(0, K, _zrow, 0)
    off = 0
    for sz in [K] * (RPT // K) + ([RPT % K] if RPT % K else []):
        pltpu.sync_copy(R.at[pl.ds(0, sz)], accum.at[pl.ds(s * RPT + off, sz)])
        off += sz
    plsc.subcore_barrier()

    def _chunk(g, _):
        ebase = wid * EPW + g * K
        pltpu.sync_copy(srcp.at[pl.ds(ebase, K)], src_idx)
        pltpu.sync_copy(dstp.at[pl.ds(ebase, K)], dst_idx)
        cp1 = pltpu.async_copy(hext.at[src_idx], S, sem1)
        cp2 = pltpu.async_copy(dstt.at[dst_idx], Dt, sem2)
        cp1.wait()
        cp2.wait()

        def _edge(k, _):
            sa = S[k, pl.ds(hc, 16)]
            dv = Dt[k, pl.ds(0, 16)]
            e16 = sa + dv
            lr = jnp.where(e16 > 0, e16, 0.2 * e16)
            mub16 = dv.at[idx_hi].get(mode="promise_in_bounds")
            wv = jnp.exp(lr - mub16)
            for j in range(nv):
                if cph == 16:
                    idxj = lanes * 0 + j
                else:
                    idxj = jnp.where(lanes < 8, 2 * j, 2 * j + 1)
                wj = wv.at[idxj].get(mode="promise_in_bounds")
                R[k, pl.ds(16 * j, 16)] = wj * S[k, pl.ds(16 * j, 16)]
            R[k, pl.ds(hc, 16)] = wv * mask_den
            return 0
        lax.fori_loop(0, K, _edge, 0)
        pltpu.sync_copy(R, accum.at[dst_idx], add=True)
        return 0

    lax.fori_loop(0, NCHUNK, _chunk, 0)
    plsc.subcore_barrier()

    off = 0
    for sz in [K] * (RPT // K) + ([RPT % K] if RPT % K else []):
        r0 = s * RPT + off
        pltpu.sync_copy(accum.at[pl.ds(r0, sz)], out.at[c, pl.ds(r0, sz)])
        off += sz


@functools.lru_cache(maxsize=None)
def _sc_edge_fn(hc):
    row = hc + 16
    mesh = plsc.VectorSubcoreMesh(core_axis_name="c", subcore_axis_name="s",
                                  num_cores=NC, num_subcores=NS)
    return pl.kernel(
        functools.partial(_sc_edge_body, hc),
        out_type=jax.ShapeDtypeStruct((NC, NR, row), jnp.float32),
        mesh=mesh,
        compiler_params=pltpu.CompilerParams(use_tc_tiling_on_sc=False),
        scratch_types=[
            pltpu.VMEM((K,), jnp.int32),
            pltpu.VMEM((K,), jnp.int32),
            pltpu.VMEM((K, row), jnp.float32),
            pltpu.VMEM((K, 16), jnp.float32),
            pltpu.VMEM((K, row), jnp.float32),
            pltpu.VMEM_SHARED((NR, row), jnp.float32),
            pltpu.SemaphoreType.DMA,
            pltpu.SemaphoreType.DMA,
        ],
    )


def _gat_layer(x, src_pad, dst_pad, W, a_s, a_d, b, hc):
    hext, dstt = _dense(x, W, _att_mat(a_s), _att_mat(a_d), hc)
    hext = jnp.concatenate([hext, jnp.zeros((1, hc + 16), jnp.float32)], axis=0)
    dstt = jnp.concatenate([dstt, jnp.zeros((1, 16), jnp.float32)], axis=0)
    raw = _sc_edge_fn(hc)(hext, dstt, src_pad, dst_pad)
    acc = raw[0, :N] + raw[1, :N]
    num = acc[:, :hc].reshape(N, H, hc // H)
    den = acc[:, hc:hc + 8]
    out = num / (den[:, :, None] + 1e-16)
    return jax.nn.elu(out.reshape(N, hc) + b)


def kernel(x, edge_index, batch, W1, a1s, a1d, b1, W2, a2s, a2d, b2,
           W3, a3s, a3d, b3, fc1_w, fc1_b, fc2_w, fc2_b):
    loop = jnp.arange(N, dtype=jnp.int32)
    fill = jnp.full((EPAD - E - N,), N, jnp.int32)
    src_pad = jnp.concatenate([edge_index[0].astype(jnp.int32), loop, fill])
    dst_pad = jnp.concatenate([edge_index[1].astype(jnp.int32), loop, fill])

    h = _gat_layer(x, src_pad, dst_pad, W1, a1s, a1d, b1, 64)
    h = _gat_layer(h, src_pad, dst_pad, W2, a2s, a2d, b2, 128)
    h = _gat_layer(h, src_pad, dst_pad, W3, a3s, a3d, b3, 128)

    sums = jax.ops.segment_sum(h, batch, num_segments=G)
    cnt = jax.ops.segment_sum(jnp.ones((N,), h.dtype), batch, num_segments=G)
    g = sums / jnp.maximum(cnt, 1.0)[:, None]
    z = jax.nn.relu(g @ fc1_w + fc1_b)
    return z @ fc2_w + fc2_b


# speedup vs baseline: 77.9342x; 1.1716x over previous
import functools

import jax
import jax.numpy as jnp
from jax import lax
from jax.experimental import pallas as pl
from jax.experimental.pallas import tpu as pltpu
from jax.experimental.pallas import tpu_sc as plsc

N = 10000
E = 320000
G = 32
H = 8

NC = 2
NS = 16
EPW = 10496
EPAD = EPW * NC * NS
NR = 10016
RPT = NR // NS


def _dense_body(hc, x_ref, w_ref, as_ref, ad_ref, hext_ref, dstt_ref):
    n = x_ref.shape[0]
    h = jnp.dot(x_ref[...], w_ref[...], preferred_element_type=jnp.float32)
    asrc = jnp.dot(h, as_ref[...], preferred_element_type=jnp.float32)
    adst = jnp.dot(h, ad_ref[...], preferred_element_type=jnp.float32)
    hext_ref[...] = jnp.concatenate(
        [h, asrc, jnp.zeros((n, 8), jnp.float32)], axis=1)
    m = asrc.max(axis=0)[None, :] + adst
    mub = jnp.where(m > 0, m, 0.2 * m)
    dstt_ref[...] = jnp.concatenate([adst, mub], axis=1)


@functools.partial(jax.jit, static_argnames=("hc",))
def _dense(x, W, As, Ad, hc):
    n = x.shape[0]
    row = hc + 16
    return pl.pallas_call(
        functools.partial(_dense_body, hc),
        out_shape=(
            jax.ShapeDtypeStruct((n, row), jnp.float32),
            jax.ShapeDtypeStruct((n, 16), jnp.float32),
        ),
    )(x, W, As, Ad)


def _att_mat(a):
    Hh, C = a.shape
    return (jnp.eye(Hh, dtype=a.dtype)[:, None, :] * a[:, :, None]).reshape(Hh * C, Hh)


def _sc_edge_body(hc, kk, hext, dstt, srcp, dstp, out,
                  si0, di0, S0, Dt0, R0, si1, di1, S1, Dt1, R1,
                  accum, sS0, sD0, sS1, sD1):
    row = hc + 16
    nchunk = EPW // kk
    nv = hc // 16
    c = lax.axis_index("c")
    s = lax.axis_index("s")
    wid = c * NS + s

    cph = hc // H
    lanes = lax.iota(jnp.int32, 16)
    idx_hi = lanes % 8 + 8
    mask_den = jnp.where(lanes < 8, 1.0, 0.0).astype(jnp.float32)

    def _zrow(k, _):
        for j in range(row // 16):
            R0[k, pl.ds(16 * j, 16)] = jnp.zeros((16,), jnp.float32)
        return 0
    lax.fori_loop(0, kk, _zrow, 0)
    off = 0
    for sz in [kk] * (RPT // kk) + ([RPT % kk] if RPT % kk else []):
        pltpu.sync_copy(R0.at[pl.ds(0, sz)], accum.at[pl.ds(s * RPT + off, sz)])
        off += sz
    plsc.subcore_barrier()

    slots = ((si0, di0, S0, Dt0, R0, sS0, sD0),
             (si1, di1, S1, Dt1, R1, sS1, sD1))

    def _issue(b, g):
        si, di, S, Dt, _, sS, sD = slots[b]
        ebase = wid * EPW + g * kk
        pltpu.sync_copy(srcp.at[pl.ds(ebase, kk)], si)
        pltpu.sync_copy(dstp.at[pl.ds(ebase, kk)], di)
        pltpu.async_copy(hext.at[si], S, sS)
        pltpu.async_copy(dstt.at[di], Dt, sD)

    _issue(0, 0)
    _issue(1, 1)

    def _pair(gg, _):
        for b in (0, 1):
            g = 2 * gg + b
            si, di, S, Dt, R, sS, sD = slots[b]
            pltpu.make_async_copy(hext.at[si], S, sS).wait()
            pltpu.make_async_copy(dstt.at[di], Dt, sD).wait()

            @plsc.parallel_loop(0, kk, unroll=2)
            def _edge(k):
                sa = S[k, pl.ds(hc, 16)]
                dv = Dt[k, pl.ds(0, 16)]
                e16 = sa + dv
                lr = jnp.where(e16 > 0, e16, 0.2 * e16)
                mub16 = dv.at[idx_hi].get(mode="promise_in_bounds")
                wv = jnp.exp(lr - mub16)
                for j in range(nv):
                    if cph == 16:
                        idxj = lanes * 0 + j
                    else:
                        idxj = jnp.where(lanes < 8, 2 * j, 2 * j + 1)
                    wj = wv.at[idxj].get(mode="promise_in_bounds")
                    R[k, pl.ds(16 * j, 16)] = wj * S[k, pl.ds(16 * j, 16)]
                R[k, pl.ds(hc, 16)] = wv * mask_den

            pltpu.sync_copy(R, accum.at[di], add=True)

            @pl.when(g + 2 < nchunk)
            def _():
                _issue(b, g + 2)
        return 0

    lax.fori_loop(0, nchunk // 2, _pair, 0)
    plsc.subcore_barrier()

    off = 0
    for sz in [kk] * (RPT // kk) + ([RPT % kk] if RPT % kk else []):
        r0 = s * RPT + off
        pltpu.sync_copy(accum.at[pl.ds(r0, sz)], out.at[c, pl.ds(r0, sz)])
        off += sz


@functools.lru_cache(maxsize=None)
def _sc_edge_fn(hc):
    row = hc + 16
    kk = 128 if hc <= 64 else 64
    mesh = plsc.VectorSubcoreMesh(core_axis_name="c", subcore_axis_name="s",
                                  num_cores=NC, num_subcores=NS)
    return pl.kernel(
        functools.partial(_sc_edge_body, hc, kk),
        out_type=jax.ShapeDtypeStruct((NC, NR, row), jnp.float32),
        mesh=mesh,
        compiler_params=pltpu.CompilerParams(use_tc_tiling_on_sc=False),
        scratch_types=[
            pltpu.VMEM((kk,), jnp.int32),
            pltpu.VMEM((kk,), jnp.int32),
            pltpu.VMEM((kk, row), jnp.float32),
            pltpu.VMEM((kk, 16), jnp.float32),
            pltpu.VMEM((kk, row), jnp.float32),
            pltpu.VMEM((kk,), jnp.int32),
            pltpu.VMEM((kk,), jnp.int32),
            pltpu.VMEM((kk, row), jnp.float32),
            pltpu.VMEM((kk, 16), jnp.float32),
            pltpu.VMEM((kk, row), jnp.float32),
            pltpu.VMEM_SHARED((NR, row), jnp.float32),
            pltpu.SemaphoreType.DMA,
            pltpu.SemaphoreType.DMA,
            pltpu.SemaphoreType.DMA,
            pltpu.SemaphoreType.DMA,
        ],
    )


def _gat_layer(x, src_pad, dst_pad, W, a_s, a_d, b, hc):
    hext, dstt = _dense(x, W, _att_mat(a_s), _att_mat(a_d), hc)
    hext = jnp.concatenate([hext, jnp.zeros((1, hc + 16), jnp.float32)], axis=0)
    dstt = jnp.concatenate([dstt, jnp.zeros((1, 16), jnp.float32)], axis=0)
    raw = _sc_edge_fn(hc)(hext, dstt, src_pad, dst_pad)
    acc = raw[0, :N] + raw[1, :N]
    num = acc[:, :hc].reshape(N, H, hc // H)
    den = acc[:, hc:hc + 8]
    out = num / (den[:, :, None] + 1e-16)
    return jax.nn.elu(out.reshape(N, hc) + b)


def kernel(x, edge_index, batch, W1, a1s, a1d, b1, W2, a2s, a2d, b2,
           W3, a3s, a3d, b3, fc1_w, fc1_b, fc2_w, fc2_b):
    loop = jnp.arange(N, dtype=jnp.int32)
    fill = jnp.full((EPAD - E - N,), N, jnp.int32)
    src_pad = jnp.concatenate([edge_index[0].astype(jnp.int32), loop, fill])
    dst_pad = jnp.concatenate([edge_index[1].astype(jnp.int32), loop, fill])

    h = _gat_layer(x, src_pad, dst_pad, W1, a1s, a1d, b1, 64)
    h = _gat_layer(h, src_pad, dst_pad, W2, a2s, a2d, b2, 128)
    h = _gat_layer(h, src_pad, dst_pad, W3, a3s, a3d, b3, 128)

    sums = jax.ops.segment_sum(h, batch, num_segments=G)
    cnt = jax.ops.segment_sum(jnp.ones((N,), h.dtype), batch, num_segments=G)
    g = sums / jnp.maximum(cnt, 1.0)[:, None]
    z = jax.nn.relu(g @ fc1_w + fc1_b)
    return z @ fc2_w + fc2_b


# speedup vs baseline: 88.3100x; 1.1331x over previous
import functools

import jax
import jax.numpy as jnp
from jax import lax
from jax.experimental import pallas as pl
from jax.experimental.pallas import tpu as pltpu
from jax.experimental.pallas import tpu_sc as plsc

N = 10000
E = 320000
G = 32
H = 8

NC = 2
NS = 16
KK = 64
NCH = 164
EPW = KK * NCH
EPAD = EPW * NC * NS
NW = NC * NS
NR = 10016
RPT = NR // NS


def _dense_body(hc, x_ref, w_ref, as_ref, ad_ref, hext_ref, dstt_ref):
    n = x_ref.shape[0]
    h = jnp.dot(x_ref[...], w_ref[...], preferred_element_type=jnp.float32)
    asrc = jnp.dot(h, as_ref[...], preferred_element_type=jnp.float32)
    adst = jnp.dot(h, ad_ref[...], preferred_element_type=jnp.float32)
    hext_ref[...] = jnp.concatenate(
        [h, asrc, jnp.zeros((n, 8), jnp.float32)], axis=1)
    m = asrc.max(axis=0)[None, :] + adst
    mub = jnp.where(m > 0, m, 0.2 * m)
    dstt_ref[...] = jnp.concatenate([adst, mub], axis=1)


@functools.partial(jax.jit, static_argnames=("hc",))
def _dense(x, W, As, Ad, hc):
    n = x.shape[0]
    row = hc + 16
    return pl.pallas_call(
        functools.partial(_dense_body, hc),
        out_shape=(
            jax.ShapeDtypeStruct((n, row), jnp.float32),
            jax.ShapeDtypeStruct((n, 16), jnp.float32),
        ),
    )(x, W, As, Ad)


def _att_mat(a):
    Hh, C = a.shape
    return (jnp.eye(Hh, dtype=a.dtype)[:, None, :] * a[:, :, None]).reshape(Hh * C, Hh)


def _sc_edge_body(hc, idx_all, hext, dstt, out,
                  I0, I1, I2, I3, S0, Dt0, R0, S1, Dt1, R1, accum,
                  sS0, sD0, sS1, sD1, sI0, sI1, sI2, sI3):
    row = hc + 16
    nv = hc // 16
    c = lax.axis_index("c")
    s = lax.axis_index("s")
    wid = c * NS + s

    cph = hc // H
    lanes = lax.iota(jnp.int32, 16)
    idx_hi = lanes % 8 + 8
    mask_den = jnp.where(lanes < 8, 1.0, 0.0).astype(jnp.float32)

    Islots = (I0, I1, I2, I3)
    Isems = (sI0, sI1, sI2, sI3)
    data = ((S0, Dt0, R0, sS0, sD0), (S1, Dt1, R1, sS1, sD1))

    def _zrow(k, _):
        for j in range(row // 16):
            R0[k, pl.ds(16 * j, 16)] = jnp.zeros((16,), jnp.float32)
        return 0
    lax.fori_loop(0, KK, _zrow, 0)
    off = 0
    for sz in [KK] * (RPT // KK) + ([RPT % KK] if RPT % KK else []):
        pltpu.sync_copy(R0.at[pl.ds(0, sz)], accum.at[pl.ds(s * RPT + off, sz)])
        off += sz
    plsc.subcore_barrier()

    def _issue_idx(isl, g):
        pltpu.async_copy(idx_all.at[wid, g], Islots[isl], Isems[isl])

    def _issue_gather(isl, b):
        S, Dt, _, sS, sD = data[b]
        pltpu.async_copy(hext.at[Islots[isl].at[0]], S, sS)
        pltpu.async_copy(dstt.at[Islots[isl].at[1]], Dt, sD)

    for g in range(4):
        _issue_idx(g, g)
    for g in range(2):
        pltpu.make_async_copy(idx_all.at[wid, g], Islots[g], Isems[g]).wait()
        _issue_gather(g, g)

    def _quad(gq, _):
        for b4 in range(4):
            g = 4 * gq + b4
            b = b4 % 2
            isl = b4
            S, Dt, R, sS, sD = data[b]
            pltpu.make_async_copy(hext.at[Islots[isl].at[0]], S, sS).wait()
            pltpu.make_async_copy(dstt.at[Islots[isl].at[1]], Dt, sD).wait()

            @plsc.parallel_loop(0, KK, unroll=2)
            def _edge(k):
                sa = S[k, pl.ds(hc, 16)]
                dv = Dt[k, pl.ds(0, 16)]
                e16 = sa + dv
                lr = jnp.where(e16 > 0, e16, 0.2 * e16)
                mub16 = dv.at[idx_hi].get(mode="promise_in_bounds")
                wv = jnp.exp(lr - mub16)
                for j in range(nv):
                    if cph == 16:
                        idxj = lanes * 0 + j
                    else:
                        idxj = jnp.where(lanes < 8, 2 * j, 2 * j + 1)
                    wj = wv.at[idxj].get(mode="promise_in_bounds")
                    R[k, pl.ds(16 * j, 16)] = wj * S[k, pl.ds(16 * j, 16)]
                R[k, pl.ds(hc, 16)] = wv * mask_den

            pltpu.sync_copy(R, accum.at[Islots[isl].at[1]], add=True)

            @pl.when(g + 2 < NCH)
            def _():
                isl2 = (b4 + 2) % 4
                pltpu.make_async_copy(
                    idx_all.at[wid, g + 2], Islots[isl2], Isems[isl2]).wait()
                _issue_gather(isl2, b)

            @pl.when(g + 4 < NCH)
            def _():
                _issue_idx(isl, g + 4)
        return 0

    lax.fori_loop(0, NCH // 4, _quad, 0)
    plsc.subcore_barrier()

    off = 0
    for sz in [KK] * (RPT // KK) + ([RPT % KK] if RPT % KK else []):
        r0 = s * RPT + off
        pltpu.sync_copy(accum.at[pl.ds(r0, sz)], out.at[c, pl.ds(r0, sz)])
        off += sz


@functools.lru_cache(maxsize=None)
def _sc_edge_fn(hc):
    row = hc + 16
    mesh = plsc.VectorSubcoreMesh(core_axis_name="c", subcore_axis_name="s",
                                  num_cores=NC, num_subcores=NS)
    return pl.kernel(
        functools.partial(_sc_edge_body, hc),
        out_type=jax.ShapeDtypeStruct((NC, NR, row), jnp.float32),
        mesh=mesh,
        compiler_params=pltpu.CompilerParams(use_tc_tiling_on_sc=False),
        scratch_types=[
            pltpu.VMEM((2, KK), jnp.int32),
            pltpu.VMEM((2, KK), jnp.int32),
            pltpu.VMEM((2, KK), jnp.int32),
            pltpu.VMEM((2, KK), jnp.int32),
            pltpu.VMEM((KK, row), jnp.float32),
            pltpu.VMEM((KK, 16), jnp.float32),
            pltpu.VMEM((KK, row), jnp.float32),
            pltpu.VMEM((KK, row), jnp.float32),
            pltpu.VMEM((KK, 16), jnp.float32),
            pltpu.VMEM((KK, row), jnp.float32),
            pltpu.VMEM_SHARED((NR, row), jnp.float32),
            pltpu.SemaphoreType.DMA,
            pltpu.SemaphoreType.DMA,
            pltpu.SemaphoreType.DMA,
            pltpu.SemaphoreType.DMA,
            pltpu.SemaphoreType.DMA,
            pltpu.SemaphoreType.DMA,
            pltpu.SemaphoreType.DMA,
            pltpu.SemaphoreType.DMA,
        ],
    )


def _gat_layer(x, idx_all, W, a_s, a_d, b, hc):
    hext, dstt = _dense(x, W, _att_mat(a_s), _att_mat(a_d), hc)
    hext = jnp.concatenate([hext, jnp.zeros((1, hc + 16), jnp.float32)], axis=0)
    dstt = jnp.concatenate([dstt, jnp.zeros((1, 16), jnp.float32)], axis=0)
    raw = _sc_edge_fn(hc)(idx_all, hext, dstt)
    acc = raw[0, :N] + raw[1, :N]
    num = acc[:, :hc].reshape(N, H, hc // H)
    den = acc[:, hc:hc + 8]
    out = num / (den[:, :, None] + 1e-16)
    return jax.nn.elu(out.reshape(N, hc) + b)


def kernel(x, edge_index, batch, W1, a1s, a1d, b1, W2, a2s, a2d, b2,
           W3, a3s, a3d, b3, fc1_w, fc1_b, fc2_w, fc2_b):
    loop = jnp.arange(N, dtype=jnp.int32)
    fill = jnp.full((EPAD - E - N,), N, jnp.int32)
    src_pad = jnp.concatenate([edge_index[0].astype(jnp.int32), loop, fill])
    dst_pad = jnp.concatenate([edge_index[1].astype(jnp.int32), loop, fill])
    idx_all = jnp.stack([src_pad.reshape(NW, NCH, KK),
                         dst_pad.reshape(NW, NCH, KK)], axis=2)

    h = _gat_layer(x, idx_all, W1, a1s, a1d, b1, 64)
    h = _gat_layer(h, idx_all, W2, a2s, a2d, b2, 128)
    h = _gat_layer(h, idx_all, W3, a3s, a3d, b3, 128)

    sums = jax.ops.segment_sum(h, batch, num_segments=G)
    cnt = jax.ops.segment_sum(jnp.ones((N,), h.dtype), batch, num_segments=G)
    g = sums / jnp.maximum(cnt, 1.0)[:, None]
    z = jax.nn.relu(g @ fc1_w + fc1_b)
    return z @ fc2_w + fc2_b


# speedup vs baseline: 91.0682x; 1.0312x over previous
import functools

import jax
import jax.numpy as jnp
from jax import lax
from jax.experimental import pallas as pl
from jax.experimental.pallas import tpu as pltpu
from jax.experimental.pallas import tpu_sc as plsc

N = 10000
E = 320000
G = 32
H = 8

NC = 2
NS = 16
KK = 64
NCH = 164
EPW = KK * NCH
EPAD = EPW * NC * NS
NW = NC * NS
NR = 10016
RPT = NR // NS


def _combine_body(hc_in, raw_ref, b_ref, xin_ref):
    raw = raw_ref[...]
    acc = raw[0] + raw[1]
    nb = acc.shape[0]
    num = acc[:, :hc_in].reshape(nb, H, hc_in // H)
    den = acc[:, hc_in:hc_in + 8]
    xin = num / (den[:, :, None] + 1e-16)
    xin = xin.reshape(nb, hc_in) + b_ref[...][None, :]
    xin_ref[...] = jnp.where(xin > 0, xin, jnp.exp(xin) - 1.0)


@functools.partial(jax.jit, static_argnames=("hc_in",))
def _combine(raw, b, hc_in):
    row_in = hc_in + 16
    nrb = NR // 4
    return pl.pallas_call(
        functools.partial(_combine_body, hc_in),
        grid=(4,),
        in_specs=[
            pl.BlockSpec((2, nrb, row_in), lambda i: (0, i, 0)),
            pl.BlockSpec((hc_in,), lambda i: (0,)),
        ],
        out_specs=pl.BlockSpec((nrb, hc_in), lambda i: (i, 0)),
        out_shape=jax.ShapeDtypeStruct((NR, hc_in), jnp.float32),
    )(raw, b)


def _dense_body0(hc, x_ref, w_ref, as_ref, ad_ref, hext_ref, dstt_ref):
    n = x_ref.shape[0]
    h = jnp.dot(x_ref[...], w_ref[...], preferred_element_type=jnp.float32)
    asrc = jnp.dot(h, as_ref[...], preferred_element_type=jnp.float32)
    adst = jnp.dot(h, ad_ref[...], preferred_element_type=jnp.float32)
    hext_ref[...] = jnp.concatenate(
        [h, asrc, jnp.zeros((n, 8), jnp.float32)], axis=1)
    m = asrc.max(axis=0)[None, :] + adst
    mub = jnp.where(m > 0, m, 0.2 * m)
    dstt_ref[...] = jnp.concatenate([adst, mub], axis=1)


@functools.partial(jax.jit, static_argnames=("hc",))
def _dense0(x, W, As, Ad, hc):
    n = x.shape[0]
    row = hc + 16
    return pl.pallas_call(
        functools.partial(_dense_body0, hc),
        out_shape=(
            jax.ShapeDtypeStruct((n, row), jnp.float32),
            jax.ShapeDtypeStruct((n, 16), jnp.float32),
        ),
    )(x, W, As, Ad)


def _head_body(h_ref, batch_ref, f1w_ref, f1b_ref, f2w_ref, f2b_ref, out_ref):
    h = h_ref[...]
    gid = lax.broadcasted_iota(jnp.int32, (N, G), 1)
    P = (batch_ref[...] == gid).astype(jnp.float32)
    sums = lax.dot_general(P, h, (((0,), (0,)), ((), ())),
                           preferred_element_type=jnp.float32)
    cnt = jnp.sum(P, axis=0)
    gm = sums / jnp.maximum(cnt, 1.0)[:, None]
    z = jnp.dot(gm, f1w_ref[...], preferred_element_type=jnp.float32)
    z = jnp.maximum(z + f1b_ref[...][None, :], 0.0)
    out_ref[...] = (jnp.dot(z, f2w_ref[...], preferred_element_type=jnp.float32)
                    + f2b_ref[...][None, :])


@jax.jit
def _head(h, batch2, f1w, f1b, f2w, f2b):
    return pl.pallas_call(
        _head_body,
        out_shape=jax.ShapeDtypeStruct((G, 1), jnp.float32),
    )(h, batch2, f1w, f1b, f2w, f2b)


def _att_mat(a):
    Hh, C = a.shape
    return (jnp.eye(Hh, dtype=a.dtype)[:, None, :] * a[:, :, None]).reshape(Hh * C, Hh)


def _sc_edge_body(hc, idx_all, hext, dstt, out,
                  I0, I1, I2, I3, S0, Dt0, R0, S1, Dt1, R1, accum,
                  sS0, sD0, sS1, sD1, sI0, sI1, sI2, sI3):
    row = hc + 16
    nv = hc // 16
    c = lax.axis_index("c")
    s = lax.axis_index("s")
    wid = c * NS + s

    cph = hc // H
    lanes = lax.iota(jnp.int32, 16)
    idx_hi = lanes % 8 + 8
    mask_den = jnp.where(lanes < 8, 1.0, 0.0).astype(jnp.float32)

    Islots = (I0, I1, I2, I3)
    Isems = (sI0, sI1, sI2, sI3)
    data = ((S0, Dt0, R0, sS0, sD0), (S1, Dt1, R1, sS1, sD1))

    def _zrow(k, _):
        for j in range(row // 16):
            R0[k, pl.ds(16 * j, 16)] = jnp.zeros((16,), jnp.float32)
        return 0
    lax.fori_loop(0, KK, _zrow, 0)
    off = 0
    for sz in [KK] * (RPT // KK) + ([RPT % KK] if RPT % KK else []):
        pltpu.sync_copy(R0.at[pl.ds(0, sz)], accum.at[pl.ds(s * RPT + off, sz)])
        off += sz
    plsc.subcore_barrier()

    def _issue_idx(isl, g):
        pltpu.async_copy(idx_all.at[wid, g], Islots[isl], Isems[isl])

    def _issue_gather(isl, b):
        S, Dt, _, sS, sD = data[b]
        pltpu.async_copy(hext.at[Islots[isl].at[0]], S, sS)
        pltpu.async_copy(dstt.at[Islots[isl].at[1]], Dt, sD)

    for g in range(4):
        _issue_idx(g, g)
    for g in range(2):
        pltpu.make_async_copy(idx_all.at[wid, g], Islots[g], Isems[g]).wait()
        _issue_gather(g, g)

    def _quad(gq, _):
        for b4 in range(4):
            g = 4 * gq + b4
            b = b4 % 2
            isl = b4
            S, Dt, R, sS, sD = data[b]
            pltpu.make_async_copy(hext.at[Islots[isl].at[0]], S, sS).wait()
            pltpu.make_async_copy(dstt.at[Islots[isl].at[1]], Dt, sD).wait()

            @plsc.parallel_loop(0, KK, unroll=2)
            def _edge(k):
                sa = S[k, pl.ds(hc, 16)]
                dv = Dt[k, pl.ds(0, 16)]
                e16 = sa + dv
                lr = jnp.where(e16 > 0, e16, 0.2 * e16)
                mub16 = dv.at[idx_hi].get(mode="promise_in_bounds")
                wv = jnp.exp(lr - mub16)
                for j in range(nv):
                    if cph == 16:
                        idxj = lanes * 0 + j
                    else:
                        idxj = jnp.where(lanes < 8, 2 * j, 2 * j + 1)
                    wj = wv.at[idxj].get(mode="promise_in_bounds")
                    R[k, pl.ds(16 * j, 16)] = wj * S[k, pl.ds(16 * j, 16)]
                R[k, pl.ds(hc, 16)] = wv * mask_den

            pltpu.sync_copy(R, accum.at[Islots[isl].at[1]], add=True)

            @pl.when(g + 2 < NCH)
            def _():
                isl2 = (b4 + 2) % 4
                pltpu.make_async_copy(
                    idx_all.at[wid, g + 2], Islots[isl2], Isems[isl2]).wait()
                _issue_gather(isl2, b)

            @pl.when(g + 4 < NCH)
            def _():
                _issue_idx(isl, g + 4)
        return 0

    lax.fori_loop(0, NCH // 4, _quad, 0)
    plsc.subcore_barrier()

    off = 0
    for sz in [KK] * (RPT // KK) + ([RPT % KK] if RPT % KK else []):
        r0 = s * RPT + off
        pltpu.sync_copy(accum.at[pl.ds(r0, sz)], out.at[c, pl.ds(r0, sz)])
        off += sz


@functools.lru_cache(maxsize=None)
def _sc_edge_fn(hc):
    row = hc + 16
    mesh = plsc.VectorSubcoreMesh(core_axis_name="c", subcore_axis_name="s",
                                  num_cores=NC, num_subcores=NS)
    return pl.kernel(
        functools.partial(_sc_edge_body, hc),
        out_type=jax.ShapeDtypeStruct((NC, NR, row), jnp.float32),
        mesh=mesh,
        compiler_params=pltpu.CompilerParams(use_tc_tiling_on_sc=False),
        scratch_types=[
            pltpu.VMEM((2, KK), jnp.int32),
            pltpu.VMEM((2, KK), jnp.int32),
            pltpu.VMEM((2, KK), jnp.int32),
            pltpu.VMEM((2, KK), jnp.int32),
            pltpu.VMEM((KK, row), jnp.float32),
            pltpu.VMEM((KK, 16), jnp.float32),
            pltpu.VMEM((KK, row), jnp.float32),
            pltpu.VMEM((KK, row), jnp.float32),
            pltpu.VMEM((KK, 16), jnp.float32),
            pltpu.VMEM((KK, row), jnp.float32),
            pltpu.VMEM_SHARED((NR, row), jnp.float32),
            pltpu.SemaphoreType.DMA,
            pltpu.SemaphoreType.DMA,
            pltpu.SemaphoreType.DMA,
            pltpu.SemaphoreType.DMA,
            pltpu.SemaphoreType.DMA,
            pltpu.SemaphoreType.DMA,
            pltpu.SemaphoreType.DMA,
            pltpu.SemaphoreType.DMA,
        ],
    )


def kernel(x, edge_index, batch, W1, a1s, a1d, b1, W2, a2s, a2d, b2,
           W3, a3s, a3d, b3, fc1_w, fc1_b, fc2_w, fc2_b):
    loop = jnp.arange(N, dtype=jnp.int32)
    fill = jnp.full((EPAD - E - N,), N, jnp.int32)
    src_pad = jnp.concatenate([edge_index[0].astype(jnp.int32), loop, fill])
    dst_pad = jnp.concatenate([edge_index[1].astype(jnp.int32), loop, fill])
    idx_all = jnp.stack([src_pad.reshape(NW, NCH, KK),
                         dst_pad.reshape(NW, NCH, KK)], axis=2)

    def tables(hext, dstt):
        hext = jnp.concatenate(
            [hext, jnp.zeros((1, hext.shape[1]), jnp.float32)], axis=0)
        dstt = jnp.concatenate([dstt, jnp.zeros((1, 16), jnp.float32)], axis=0)
        return hext, dstt

    hext, dstt = _dense0(x, W1, _att_mat(a1s), _att_mat(a1d), 64)
    raw = _sc_edge_fn(64)(idx_all, *tables(hext, dstt))
    xin = _combine(raw, b1, 64)
    hext, dstt = _dense0(xin, W2, _att_mat(a2s), _att_mat(a2d), 128)
    raw = _sc_edge_fn(128)(idx_all, hext, dstt)
    xin = _combine(raw, b2, 128)
    hext, dstt = _dense0(xin, W3, _att_mat(a3s), _att_mat(a3d), 128)
    raw = _sc_edge_fn(128)(idx_all, hext, dstt)
    h3 = _combine(raw, b3, 128)[:N]

    batch2 = batch.astype(jnp.int32).reshape(N, 1)
    return _head(h3, batch2, fc1_w, fc1_b, fc2_w, fc2_b)
